# baseline (speedup 1.0000x reference)
import functools

import jax
import jax.numpy as jnp
from jax import lax
from jax.experimental import pallas as pl
from jax.experimental.pallas import tpu as pltpu
from jax.experimental.pallas import tpu_sc as plsc

N = 50000
D = 256
G = 256
L = 16
NW = 32
RB = 40
NBLK = N // RB
NBUF = 6
NC = D // L
_EXTRA = NBLK % NW
_BASE_BLKS = NBLK // NW

_mesh = plsc.VectorSubcoreMesh(core_axis_name="c", subcore_axis_name="s")

_NEG = float("-inf")


@functools.partial(
    pl.kernel,
    out_type=jax.ShapeDtypeStruct((NW, G * D), jnp.float32),
    mesh=_mesh,
    compiler_params=pltpu.CompilerParams(needs_layout_passes=False),
    scratch_types=(
        [pltpu.VMEM((RB, D), jnp.float32)] * NBUF
        + [pltpu.VMEM((G * D,), jnp.float32)]
        + [pltpu.VMEM((128,), jnp.int32)] * NBUF
        + [pltpu.SemaphoreType.DMA] * (2 * NBUF)
    ),
)
def _seg_max_partial(feat_hbm, aug_hbm, part_hbm, *scr):
    rowbufs = scr[0:NBUF]
    accf = scr[NBUF]
    augbufs = scr[NBUF + 1 : 2 * NBUF + 1]
    semfs = scr[2 * NBUF + 1 : 3 * NBUF + 1]
    semas = scr[3 * NBUF + 1 : 4 * NBUF + 1]
    w = lax.axis_index("s") * 2 + lax.axis_index("c")
    iota = lax.broadcasted_iota(jnp.int32, (L,), 0)
    neg = jnp.full((L,), _NEG, jnp.float32)

    def init_body(i, carry):
        for u in range(4):
            accf[pl.ds((4 * i + u) * L, L)] = neg
        return carry

    lax.fori_loop(0, G * D // L // 4, init_body, 0)

    pad = jnp.full((L,), -1, jnp.int32)
    for sb in augbufs:
        for i in range(RB // L, 128 // L):
            sb[pl.ds(i * L, L)] = pad
        if RB % L:
            sb[pl.ds(RB, L)] = pad

    nb = jnp.where(w < _EXTRA, _BASE_BLKS + 1, _BASE_BLKS)

    def start(k, par):
        b = w + NW * k
        pltpu.async_copy(feat_hbm.at[pl.ds(b * RB, RB), :], rowbufs[par], semfs[par])
        pltpu.async_copy(
            aug_hbm.at[pl.ds(b * RB, RB)], augbufs[par].at[pl.ds(0, RB)], semas[par]
        )

    def wait(k, par):
        b = w + NW * k
        pltpu.make_async_copy(
            feat_hbm.at[pl.ds(b * RB, RB), :], rowbufs[par], semfs[par]
        ).wait()
        pltpu.make_async_copy(
            aug_hbm.at[pl.ds(b * RB, RB)], augbufs[par].at[pl.ds(0, RB)], semas[par]
        ).wait()

    def compute(k, par):
        rowbuf = rowbufs[par]
        augbuf = augbufs[par]

        def run_cond(r):
            return r < RB

        def run_body(r):
            sv = plsc.load_gather(augbuf, [lax.broadcast(r, (L,))])
            idx0 = jnp.minimum(sv, G - 1) * D + iota

            def w_cond(c):
                pos, ew = c
                return (ew == L) & (pos < RB)

            def w_body(c):
                pos, _ = c
                wv = augbuf[pl.ds(pos, L)]
                ew = jnp.max(jnp.where(wv == sv, iota + 1, jnp.int32(0)))
                return (jnp.minimum(pos + ew, RB), ew)

            e, _ = lax.while_loop(w_cond, w_body, (r, jnp.int32(L)))
            accs = []
            for c in range(NC):
                old = plsc.load_gather(accf, [idx0 + (c * L)])
                accs.append(jnp.maximum(old, rowbuf[r, pl.ds(c * L, L)]))

            def row_body(rr, accs):
                return tuple(
                    jnp.maximum(accs[c], rowbuf[rr, pl.ds(c * L, L)])
                    for c in range(NC)
                )

            accs = lax.fori_loop(r + 1, e, row_body, tuple(accs))
            for c in range(NC):
                plsc.store_scatter(accf, [idx0 + (c * L)], accs[c])
            return e

        lax.while_loop(run_cond, run_body, jnp.int32(0))

    for k0 in range(NBUF):
        @pl.when(nb > k0)
        def _(k0=k0):
            start(k0, k0)

    def ring_body(j, carry):
        for par in range(NBUF):
            k = NBUF * j + par

            @pl.when(k < nb)
            def _():
                wait(k, par)
                compute(k, par)

                @pl.when(k + NBUF < nb)
                def _():
                    start(k + NBUF, par)

        return carry

    lax.fori_loop(0, (_BASE_BLKS + NBUF) // NBUF, ring_body, 0)
    pltpu.sync_copy(accf, part_hbm.at[w])


_SEG_PER_W = G // NW
_CH = _SEG_PER_W * D


@functools.partial(
    pl.kernel,
    out_type=jax.ShapeDtypeStruct((G * D,), jnp.float32),
    mesh=_mesh,
    compiler_params=pltpu.CompilerParams(needs_layout_passes=False),
    scratch_types=[
        pltpu.VMEM((NW, _CH), jnp.float32),
        pltpu.VMEM((_CH,), jnp.float32),
    ],
)
def _combine(part_hbm, out_hbm, buf, acc):
    w = lax.axis_index("s") * 2 + lax.axis_index("c")
    pltpu.sync_copy(part_hbm.at[:, pl.ds(w * _CH, _CH)], buf)

    def m_body(i, carry):
        sl = pl.ds(i * L, L)
        m = jnp.maximum(buf[0, sl], buf[1, sl])
        for t in range(2, NW):
            m = jnp.maximum(m, buf[t, sl])
        acc[sl] = m
        return carry

    lax.fori_loop(0, _CH // L, m_body, 0)
    pltpu.sync_copy(acc, out_hbm.at[pl.ds(w * _CH, _CH)])


def kernel(feat, segment_ids, num_graphs):
    part = _seg_max_partial(feat, segment_ids.astype(jnp.int32))
    outf = _combine(part)
    return outf.reshape(G, D)


# speedup vs baseline: 1.0939x; 1.0939x over previous
import functools

import jax
import jax.numpy as jnp
from jax import lax
from jax.experimental import pallas as pl
from jax.experimental.pallas import tpu as pltpu
from jax.experimental.pallas import tpu_sc as plsc

N = 50000
D = 256
G = 256
L = 16
NW = 32
RB = 80
NBLK = N // RB
NC = D // L
_EXTRA = NBLK % NW
_BASE_BLKS = NBLK // NW

_mesh = plsc.VectorSubcoreMesh(core_axis_name="c", subcore_axis_name="s")

_NEG = float("-inf")


@functools.partial(
    pl.kernel,
    out_type=(
        jax.ShapeDtypeStruct((2, 16, G, D), jnp.float32),
        jax.ShapeDtypeStruct((2, G, D), jnp.float32),
    ),
    mesh=_mesh,
    compiler_params=pltpu.CompilerParams(needs_layout_passes=False),
    scratch_types=[
        pltpu.VMEM((RB, D), jnp.float32),
        pltpu.VMEM((RB, D), jnp.float32),
        pltpu.VMEM((RB, D), jnp.float32),
        pltpu.VMEM((G, D), jnp.float32),
        pltpu.VMEM((128,), jnp.int32),
        pltpu.VMEM((128,), jnp.int32),
        pltpu.VMEM((128,), jnp.int32),
        pltpu.SemaphoreType.DMA,
        pltpu.SemaphoreType.DMA,
        pltpu.SemaphoreType.DMA,
        pltpu.SemaphoreType.DMA,
        pltpu.SemaphoreType.DMA,
        pltpu.SemaphoreType.DMA,
    ],
)
def _seg_max_partial(
    feat_hbm, aug_hbm, stage_hbm, psc_hbm, rowb0, rowb1, rowb2, accf,
    augb0, augb1, augb2, semf0, semf1, semf2, sema0, sema1, sema2,
):
    sid = lax.axis_index("s")
    cid = lax.axis_index("c")
    w = sid * 2 + cid
    iota = lax.broadcasted_iota(jnp.int32, (L,), 0)
    neg = jnp.full((L,), _NEG, jnp.float32)
    rowbufs = (rowb0, rowb1, rowb2)
    augbufs = (augb0, augb1, augb2)
    semfs = (semf0, semf1, semf2)
    semas = (sema0, sema1, sema2)

    def init_body(r, carry):
        for c in range(NC):
            accf[r, pl.ds(c * L, L)] = neg
        return carry

    lax.fori_loop(0, G, init_body, 0)

    pad = jnp.full((L,), -1, jnp.int32)
    for sb in augbufs:
        for i in range(RB // L, 128 // L):
            sb[pl.ds(i * L, L)] = pad

    nb = jnp.where(w < _EXTRA, _BASE_BLKS + 1, _BASE_BLKS)

    def start(k, par):
        b = w + NW * k
        pltpu.async_copy(feat_hbm.at[pl.ds(b * RB, RB), :], rowbufs[par], semfs[par])
        pltpu.async_copy(
            aug_hbm.at[pl.ds(b * RB, RB)], augbufs[par].at[pl.ds(0, RB)], semas[par]
        )

    def wait(k, par):
        b = w + NW * k
        pltpu.make_async_copy(
            feat_hbm.at[pl.ds(b * RB, RB), :], rowbufs[par], semfs[par]
        ).wait()
        pltpu.make_async_copy(
            aug_hbm.at[pl.ds(b * RB, RB)], augbufs[par].at[pl.ds(0, RB)], semas[par]
        ).wait()

    def compute(k, par):
        rowbuf = rowbufs[par]
        augbuf = augbufs[par]

        def run_cond(r):
            return r < RB

        def run_body(r):
            sv = plsc.load_gather(augbuf, [lax.broadcast(r, (L,))])
            row = jnp.minimum(sv, G - 1)

            def w_cond(c):
                pos, ew = c
                return (ew == L) & (pos < RB)

            def w_body(c):
                pos, _ = c
                wv = augbuf[pl.ds(pos, L)]
                ew = jnp.max(jnp.where(wv == sv, iota + 1, jnp.int32(0)))
                return (jnp.minimum(pos + ew, RB), ew)

            e, _ = lax.while_loop(w_cond, w_body, (r, jnp.int32(L)))
            accs = []
            for c in range(NC):
                old = plsc.load_gather(accf, [row, iota + (c * L)])
                accs.append(jnp.maximum(old, rowbuf[r, pl.ds(c * L, L)]))

            def row_body(rr, accs):
                return tuple(
                    jnp.maximum(accs[c], rowbuf[rr, pl.ds(c * L, L)])
                    for c in range(NC)
                )

            accs = lax.fori_loop(r + 1, e, row_body, tuple(accs))
            for c in range(NC):
                plsc.store_scatter(accf, [row, iota + (c * L)], accs[c])
            return e

        lax.while_loop(run_cond, run_body, jnp.int32(0))

    start(0, 0)

    @pl.when(nb > 1)
    def _():
        start(1, 1)

    @pl.when(nb > 2)
    def _():
        start(2, 2)

    def trio_body(j, carry):
        for par in range(3):
            k = 3 * j + par

            @pl.when(k < nb)
            def _():
                wait(k, par)
                compute(k, par)

                @pl.when(k + 3 < nb)
                def _():
                    start(k + 3, par)

        return carry

    lax.fori_loop(0, (_BASE_BLKS + 3) // 3, trio_body, 0)

    SR = G // 16
    for r in range(16):
        pltpu.async_copy(
            accf.at[pl.ds(r * SR, SR), :],
            stage_hbm.at[cid, r, pl.ds(sid * SR, SR), :],
            semf0,
        )
    for r in range(16):
        pltpu.make_async_copy(
            accf.at[pl.ds(r * SR, SR), :],
            stage_hbm.at[cid, r, pl.ds(sid * SR, SR), :],
            semf0,
        ).wait()
    plsc.subcore_barrier()

    def rd_copy(j):
        return pltpu.make_async_copy(
            stage_hbm.at[cid, sid, pl.ds(j * 4 * SR, 4 * SR), :],
            rowbufs[j % 2].at[pl.ds(0, 4 * SR), :],
            semfs[1 + (j % 2)],
        )

    rd_copy(0).start()
    rd_copy(1).start()
    for j in range(4):
        rd_copy(j).wait()
        if j + 2 < 4:
            rd_copy(j + 2).start()
        buf = rowbufs[j % 2]

        def red_body(rr, carry):
            for c in range(NC):
                sl = pl.ds(c * L, L)
                m = jnp.maximum(buf[rr, sl], buf[SR + rr, sl])
                m = jnp.maximum(m, buf[2 * SR + rr, sl])
                m = jnp.maximum(m, buf[3 * SR + rr, sl])
                if j > 0:
                    m = jnp.maximum(m, rowb2[rr, sl])
                rowb2[rr, sl] = m
            return carry

        lax.fori_loop(0, SR, red_body, 0)
    pltpu.sync_copy(
        rowb2.at[pl.ds(0, SR), :], psc_hbm.at[cid, pl.ds(sid * SR, SR), :]
    )


def kernel(feat, segment_ids, num_graphs):
    _, psc = _seg_max_partial(feat, segment_ids.astype(jnp.int32))
    return jnp.maximum(psc[0], psc[1])
